# baseline (speedup 1.0000x reference)
import functools

import jax
import jax.numpy as jnp
from jax import lax
from jax.experimental import pallas as pl
from jax.experimental.pallas import tpu as pltpu
from jax.experimental.pallas import tpu_sc as plsc

B, S, D, E, F, K = 1, 2048, 768, 8, 1536, 2
N = S * K
T = 128
MAX_TILES = N // T + E
NSLOT = MAX_TILES * T

NW = 32
TPW = S // NW
APW = N // NW
DI = D // 2

_sc_mesh = plsc.VectorSubcoreMesh(core_axis_name="c", subcore_axis_name="s")


def _sc_wid():
    return lax.axis_index("s") * 2 + lax.axis_index("c")


def _dispatch_scatter(x_i32, pos_even, pos_odd):
    @functools.partial(
        pl.kernel, mesh=_sc_mesh,
        out_type=jax.ShapeDtypeStruct((NSLOT, DI), jnp.int32),
        scratch_types=[
            pltpu.VMEM((TPW,), jnp.int32),
            pltpu.VMEM((TPW,), jnp.int32),
            pltpu.VMEM((TPW, DI), jnp.int32),
            pltpu.SemaphoreType.DMA,
        ])
    def k(x_hbm, pe_hbm, po_hbm, xs_hbm, ie_v, io_v, rows_v, sem):
        base = _sc_wid() * TPW
        pltpu.sync_copy(pe_hbm.at[pl.ds(base, TPW)], ie_v)
        pltpu.sync_copy(po_hbm.at[pl.ds(base, TPW)], io_v)
        pltpu.async_copy(x_hbm.at[pl.ds(base, TPW)], rows_v, sem).wait()
        pltpu.sync_copy(rows_v, xs_hbm.at[ie_v])
        pltpu.sync_copy(rows_v, xs_hbm.at[io_v])

    return k(x_i32, pos_even, pos_odd)


def _combine_gather(y_i32, pos):
    @functools.partial(
        pl.kernel, mesh=_sc_mesh,
        out_type=jax.ShapeDtypeStruct((N, DI), jnp.int32),
        scratch_types=[
            pltpu.VMEM((APW,), jnp.int32),
            pltpu.VMEM((APW, DI), jnp.int32),
            pltpu.SemaphoreType.DMA,
        ])
    def k(y_hbm, pos_hbm, ys_hbm, idx_v, rows_v, sem):
        base = _sc_wid() * APW
        pltpu.sync_copy(pos_hbm.at[pl.ds(base, APW)], idx_v)
        pltpu.async_copy(y_hbm.at[idx_v], rows_v, sem).wait()
        pltpu.sync_copy(rows_v, ys_hbm.at[pl.ds(base, APW)])

    return k(y_i32, pos)


def _ffn_body(te_ref, na_ref, xs_ref, w1_ref, b1_ref, w2_ref, b2_ref, y_ref):
    i = pl.program_id(0)

    @pl.when(i < na_ref[0])
    def _():
        h = jnp.dot(xs_ref[...], w1_ref[0], preferred_element_type=jnp.float32)
        h = h + b1_ref[0, 0]
        h = h * 0.5 * (1.0 + jax.lax.erf(h * 0.7071067811865476))
        y = jnp.dot(h.astype(jnp.bfloat16), w2_ref[0],
                    preferred_element_type=jnp.float32)
        y_ref[...] = (y + b2_ref[0, 0]).astype(jnp.bfloat16)


def _clamp(i, na_ref):
    return jnp.minimum(i, na_ref[0] - 1)


def _grouped_ffn(xs, w1, b1, w2, b2, te, na):
    grid_spec = pltpu.PrefetchScalarGridSpec(
        num_scalar_prefetch=2,
        grid=(MAX_TILES,),
        in_specs=[
            pl.BlockSpec((T, D), lambda i, te, na: (_clamp(i, na), 0)),
            pl.BlockSpec((1, D, F), lambda i, te, na: (te[_clamp(i, na)], 0, 0)),
            pl.BlockSpec((1, 1, F), lambda i, te, na: (te[_clamp(i, na)], 0, 0)),
            pl.BlockSpec((1, F, D), lambda i, te, na: (te[_clamp(i, na)], 0, 0)),
            pl.BlockSpec((1, 1, D), lambda i, te, na: (te[_clamp(i, na)], 0, 0)),
        ],
        out_specs=pl.BlockSpec((T, D), lambda i, te, na: (_clamp(i, na), 0)),
    )
    return pl.pallas_call(
        _ffn_body,
        grid_spec=grid_spec,
        out_shape=jax.ShapeDtypeStruct((NSLOT, D), jnp.bfloat16),
    )(te, na, xs, w1, b1, w2, b2)


def kernel(x, Wr, br, W1, b1, W2, b2):
    x2 = x.reshape(S, D)

    logits = jnp.einsum('bsd,de->bse', x, Wr) + br
    probs = jax.nn.softmax(logits, axis=-1)
    tkp, tki = jax.lax.top_k(probs, K)
    gates = tkp / jnp.sum(tkp, axis=-1, keepdims=True)

    e_flat = tki.reshape(N)
    onehot = (e_flat[:, None] == jnp.arange(E)[None, :]).astype(jnp.int32)
    csum = jnp.cumsum(onehot, axis=0)
    counts = csum[-1]
    rank = jnp.sum(onehot * (csum - onehot), axis=1)
    padded = ((counts + T - 1) // T) * T
    ends = jnp.cumsum(padded)
    offs = ends - padded
    pos = jnp.sum(onehot * offs[None, :], axis=1) + rank
    na = (ends[-1:] // T).astype(jnp.int32)
    tile_start = jnp.arange(MAX_TILES, dtype=jnp.int32) * T
    te = jnp.minimum((tile_start[:, None] >= ends[None, :]).sum(axis=1),
                     E - 1).astype(jnp.int32)

    x_i32 = lax.bitcast_convert_type(
        x2.astype(jnp.bfloat16).reshape(S, DI, 2), jnp.int32)
    posr = pos.reshape(S, K)
    xs_i32 = _dispatch_scatter(x_i32, posr[:, 0], posr[:, 1])
    xs = lax.bitcast_convert_type(xs_i32, jnp.bfloat16).reshape(NSLOT, D)
    y = _grouped_ffn(xs, W1.astype(jnp.bfloat16), b1.reshape(E, 1, F),
                     W2.astype(jnp.bfloat16), b2.reshape(E, 1, D), te, na)
    y_i32 = lax.bitcast_convert_type(y.reshape(NSLOT, DI, 2), jnp.int32)
    ys_i32 = _combine_gather(y_i32, pos)
    ys = lax.bitcast_convert_type(ys_i32, jnp.bfloat16).reshape(N, D)
    g = gates.reshape(S, K)
    out = (ys.reshape(S, K, D).astype(jnp.float32) * g[:, :, None]).sum(axis=1)
    return out.reshape(B, S, D)


# speedup vs baseline: 5.6626x; 5.6626x over previous
import jax
import jax.numpy as jnp
from jax.experimental import pallas as pl
from jax.experimental.pallas import tpu as pltpu

B, S, D, E, F, K = 1, 2048, 768, 8, 1536, 2
N = S * K
T = 128
MAX_TILES = N // T + E
NSLOT = MAX_TILES * T


def _ffn_body(te_ref, na_ref, xs_ref, w1_hbm, b1_ref, w2_hbm, b2_ref, y_ref,
              w1v, w2v, sem):
    i = pl.program_id(0)

    @pl.when(i == 0)
    def _():
        c1 = pltpu.make_async_copy(w1_hbm, w1v, sem)
        c1.start()
        c1.wait()
        c2 = pltpu.make_async_copy(w2_hbm, w2v, sem)
        c2.start()
        c2.wait()

    @pl.when(i < na_ref[0])
    def _():
        e = te_ref[i]
        h = jnp.dot(xs_ref[...], w1v[e], preferred_element_type=jnp.float32)
        h = h + b1_ref[e]
        h = h * 0.5 * (1.0 + jax.lax.erf(h * 0.7071067811865476))
        y = jnp.dot(h.astype(jnp.bfloat16), w2v[e],
                    preferred_element_type=jnp.float32)
        y_ref[...] = y + b2_ref[e]


def _clamp(i, na_ref):
    return jnp.minimum(i, na_ref[0] - 1)


def _grouped_ffn(xs, w1, b1, w2, b2, te, na):
    grid_spec = pltpu.PrefetchScalarGridSpec(
        num_scalar_prefetch=2,
        grid=(MAX_TILES,),
        in_specs=[
            pl.BlockSpec((T, D), lambda i, te, na: (_clamp(i, na), 0)),
            pl.BlockSpec(memory_space=pl.ANY),
            pl.BlockSpec((E, 1, F), lambda i, te, na: (0, 0, 0)),
            pl.BlockSpec(memory_space=pl.ANY),
            pl.BlockSpec((E, 1, D), lambda i, te, na: (0, 0, 0)),
        ],
        out_specs=pl.BlockSpec((T, D), lambda i, te, na: (_clamp(i, na), 0)),
        scratch_shapes=[
            pltpu.VMEM((E, D, F), jnp.bfloat16),
            pltpu.VMEM((E, F, D), jnp.bfloat16),
            pltpu.SemaphoreType.DMA,
        ],
    )
    return pl.pallas_call(
        _ffn_body,
        grid_spec=grid_spec,
        out_shape=jax.ShapeDtypeStruct((NSLOT, D), jnp.float32),
    )(te, na, xs, w1, b1, w2, b2)


def kernel(x, Wr, br, W1, b1, W2, b2):
    x2 = x.reshape(S, D)

    logits = jnp.einsum('bsd,de->bse', x, Wr) + br
    probs = jax.nn.softmax(logits, axis=-1)
    tkp, tki = jax.lax.top_k(probs, K)
    gates = tkp / jnp.sum(tkp, axis=-1, keepdims=True)

    e_flat = tki.reshape(N)
    onehot = (e_flat[:, None] == jnp.arange(E)[None, :]).astype(jnp.int32)
    csum = jnp.cumsum(onehot, axis=0)
    counts = csum[-1]
    rank = jnp.take_along_axis(csum - onehot, e_flat[:, None], axis=1)[:, 0]
    padded = ((counts + T - 1) // T) * T
    ends = jnp.cumsum(padded)
    offs = ends - padded
    pos = offs[e_flat] + rank
    tok = jnp.arange(N, dtype=jnp.int32) // K
    row_ids = jnp.zeros((NSLOT,), jnp.int32).at[pos].add(tok)
    na = (ends[-1:] // T).astype(jnp.int32)
    tile_start = jnp.arange(MAX_TILES, dtype=jnp.int32) * T
    te = jnp.minimum((tile_start[:, None] >= ends[None, :]).sum(axis=1),
                     E - 1).astype(jnp.int32)

    xs = jnp.take(x2.astype(jnp.bfloat16), row_ids, axis=0)
    y = _grouped_ffn(xs, W1.astype(jnp.bfloat16), b1.reshape(E, 1, F),
                     W2.astype(jnp.bfloat16), b2.reshape(E, 1, D), te, na)
    posr = pos.reshape(S, K)
    g = gates.reshape(S, K)
    out = (jnp.take(y, posr[:, 0], axis=0) * g[:, :1]
           + jnp.take(y, posr[:, 1], axis=0) * g[:, 1:])
    return out.reshape(B, S, D)


# speedup vs baseline: 6.0955x; 1.0764x over previous
import jax
import jax.numpy as jnp
from jax.experimental import pallas as pl
from jax.experimental.pallas import tpu as pltpu

B, S, D, E, F, K = 1, 2048, 768, 8, 1536, 2
N = S * K
T = 128
MAX_TILES = N // T + E
NSLOT = MAX_TILES * T


def _ffn_body(te_ref, na_ref, xs_ref, w1_hbm, b1_ref, w2_hbm, b2_ref, y_ref,
              w1v, w2v, sem):
    i = pl.program_id(0)

    @pl.when(i == 0)
    def _():
        c1 = pltpu.make_async_copy(w1_hbm, w1v, sem)
        c1.start()
        c1.wait()
        c2 = pltpu.make_async_copy(w2_hbm, w2v, sem)
        c2.start()
        c2.wait()

    @pl.when(i < na_ref[0])
    def _():
        e = te_ref[i]
        h = jnp.dot(xs_ref[...], w1v[e], preferred_element_type=jnp.float32)
        h = h + b1_ref[e]
        h = h * 0.5 * (1.0 + jax.lax.erf(h * 0.7071067811865476))
        y = jnp.dot(h.astype(jnp.bfloat16), w2v[e],
                    preferred_element_type=jnp.float32)
        y_ref[...] = y + b2_ref[e]


CH = 8
CHS = N // CH


def _meta_body(e_ref, pos_ref, te_ref, na_ref):
    ecol = e_ref[...]
    lane = jax.lax.broadcasted_iota(jnp.int32, (1, E), 1)
    oh = (ecol == lane).astype(jnp.float32)
    ohr = oh.reshape(CH, CHS, E)
    counts_c = ohr.sum(axis=1)
    r8 = jax.lax.broadcasted_iota(jnp.int32, (CH, CH), 0)
    c8 = jax.lax.broadcasted_iota(jnp.int32, (CH, CH), 1)
    l8 = (r8 > c8).astype(jnp.float32)
    pref = jnp.dot(l8, counts_c, preferred_element_type=jnp.float32)
    rN = jax.lax.broadcasted_iota(jnp.int32, (CHS, CHS), 0)
    cN = jax.lax.broadcasted_iota(jnp.int32, (CHS, CHS), 1)
    lN = (rN > cN).astype(jnp.float32)
    ranks = [jnp.dot(lN, ohr[c], preferred_element_type=jnp.float32)
             for c in range(CH)]
    rank_mat = jnp.stack(ranks, axis=0) + pref[:, None, :]
    rank = (ohr * rank_mat).sum(axis=2).reshape(N, 1)
    counts = counts_c.sum(axis=0, keepdims=True).astype(jnp.int32)
    padded = ((counts + T - 1) // T) * T
    u8 = (r8 <= c8).astype(jnp.float32)
    ends = jnp.dot(padded.astype(jnp.float32), u8,
                   preferred_element_type=jnp.float32).astype(jnp.int32)
    offs = ends - padded
    posf = (oh * offs.astype(jnp.float32)).sum(axis=1, keepdims=True) + rank
    pos_ref[...] = posf.astype(jnp.int32)
    ts = jax.lax.broadcasted_iota(jnp.int32, (MAX_TILES, 1), 0) * T
    te_ref[...] = jnp.minimum((ts >= ends).astype(jnp.int32).sum(
        axis=1, keepdims=True), E - 1)
    na_ref[...] = ends[:, E - 1:] // T


def _routing_meta(e_flat):
    return pl.pallas_call(
        _meta_body,
        out_shape=(
            jax.ShapeDtypeStruct((N, 1), jnp.int32),
            jax.ShapeDtypeStruct((MAX_TILES, 1), jnp.int32),
            jax.ShapeDtypeStruct((1, 1), jnp.int32),
        ),
    )(e_flat.reshape(N, 1))


def _clamp(i, na_ref):
    return jnp.minimum(i, na_ref[0] - 1)


def _grouped_ffn(xs, w1, b1, w2, b2, te, na):
    grid_spec = pltpu.PrefetchScalarGridSpec(
        num_scalar_prefetch=2,
        grid=(MAX_TILES,),
        in_specs=[
            pl.BlockSpec((T, D), lambda i, te, na: (_clamp(i, na), 0)),
            pl.BlockSpec(memory_space=pl.ANY),
            pl.BlockSpec((E, 1, F), lambda i, te, na: (0, 0, 0)),
            pl.BlockSpec(memory_space=pl.ANY),
            pl.BlockSpec((E, 1, D), lambda i, te, na: (0, 0, 0)),
        ],
        out_specs=pl.BlockSpec((T, D), lambda i, te, na: (_clamp(i, na), 0)),
        scratch_shapes=[
            pltpu.VMEM((E, D, F), jnp.bfloat16),
            pltpu.VMEM((E, F, D), jnp.bfloat16),
            pltpu.SemaphoreType.DMA,
        ],
    )
    return pl.pallas_call(
        _ffn_body,
        grid_spec=grid_spec,
        out_shape=jax.ShapeDtypeStruct((NSLOT, D), jnp.float32),
    )(te, na, xs, w1, b1, w2, b2)


def kernel(x, Wr, br, W1, b1, W2, b2):
    x2 = x.reshape(S, D)

    logits = jnp.einsum('bsd,de->bse', x, Wr) + br
    probs = jax.nn.softmax(logits, axis=-1)
    tkp, tki = jax.lax.top_k(probs, K)
    gates = tkp / jnp.sum(tkp, axis=-1, keepdims=True)

    e_flat = tki.reshape(N)
    pos2, te2, na2 = _routing_meta(e_flat)
    pos, te, na = pos2[:, 0], te2[:, 0], na2[0]
    tok = jnp.arange(N, dtype=jnp.int32) // K
    row_ids = jnp.zeros((NSLOT,), jnp.int32).at[pos].add(tok)

    xs = jnp.take(x2.astype(jnp.bfloat16), row_ids, axis=0)
    y = _grouped_ffn(xs, W1.astype(jnp.bfloat16), b1.reshape(E, 1, F),
                     W2.astype(jnp.bfloat16), b2.reshape(E, 1, D), te, na)
    posr = pos.reshape(S, K)
    g = gates.reshape(S, K)
    out = (jnp.take(y, posr[:, 0], axis=0) * g[:, :1]
           + jnp.take(y, posr[:, 1], axis=0) * g[:, 1:])
    return out.reshape(B, S, D)
